# Initial kernel scaffold; baseline (speedup 1.0000x reference)
import jax
import jax.numpy as jnp
from jax.experimental import pallas as pl


def kernel(x, edge_index, W1a, b1a, W2a, b2a, W1b, b1b, W2b, b2b, gamma, beta):
    raise NotImplementedError("write your pallas kernel here")


# speedup vs baseline: 6.2256x; 6.2256x over previous
import functools

import jax
import jax.numpy as jnp
from jax import lax
from jax.experimental import pallas as pl
from jax.experimental.pallas import tpu as pltpu
from jax.experimental.pallas import tpu_sc as plsc

N = 10000
E = 320000
D = 128

NC = 2
NS = 16
EDGES_PER_SUB = E // NS
BLK = 200
N_BLKS = EDGES_PER_SUB // BLK
NPAD = 10240
ROWS_PER_SUB = NPAD // NS


def _sc_agg(x, src, dst, zeros):
    mesh = plsc.VectorSubcoreMesh(core_axis_name="c", subcore_axis_name="s")

    @functools.partial(
        pl.kernel,
        out_type=[
            jax.ShapeDtypeStruct((NPAD, D), jnp.float32),
            jax.ShapeDtypeStruct((NPAD, D), jnp.float32),
        ],
        mesh=mesh,
        scratch_types=[
            pltpu.VMEM((BLK,), jnp.int32),
            pltpu.VMEM((BLK,), jnp.int32),
            pltpu.VMEM((BLK, D), jnp.float32),
            pltpu.VMEM_SHARED((NPAD, D), jnp.float32),
            pltpu.SemaphoreType.DMA,
        ],
    )
    def agg_kernel(x_hbm, src_hbm, dst_hbm, zeros_hbm, agg1_hbm, agg2_hbm,
                   gidx_v, sidx_v, rows_v, acc_sp, sem):
        c = lax.axis_index("c")
        s = lax.axis_index("s")

        pltpu.sync_copy(zeros_hbm.at[pl.ds(s * ROWS_PER_SUB, ROWS_PER_SUB)],
                        acc_sp.at[pl.ds(s * ROWS_PER_SUB, ROWS_PER_SUB)])
        plsc.subcore_barrier()

        def run_direction(gather_hbm, scatter_hbm, out_hbm):
            base = s * EDGES_PER_SUB

            def body(i, carry):
                off = base + i * BLK
                pltpu.sync_copy(gather_hbm.at[pl.ds(off, BLK)], gidx_v)
                pltpu.sync_copy(scatter_hbm.at[pl.ds(off, BLK)], sidx_v)
                pltpu.async_copy(x_hbm.at[gidx_v], rows_v, sem).wait()
                pltpu.sync_copy(rows_v, acc_sp.at[sidx_v], add=True)
                return carry

            lax.fori_loop(0, N_BLKS, body, 0)
            plsc.subcore_barrier()
            pltpu.sync_copy(acc_sp.at[pl.ds(s * ROWS_PER_SUB, ROWS_PER_SUB)],
                            out_hbm.at[pl.ds(s * ROWS_PER_SUB, ROWS_PER_SUB)])

        @pl.when(c == 0)
        def _():
            run_direction(src_hbm, dst_hbm, agg1_hbm)

        @pl.when(c == 1)
        def _():
            run_direction(dst_hbm, src_hbm, agg2_hbm)

    return agg_kernel(x, src, dst, zeros)


def _tc_mlp_bn(x, agg1, agg2, W1a, b1a, W2a, b2a, W1b, b1b, W2b, b2b,
               gamma, beta):
    def body(x_ref, a1_ref, a2_ref, w1a, b1a_, w2a, b2a_, w1b, b1b_, w2b,
             b2b_, g_ref, be_ref, o_ref):
        h1 = x_ref[...] + a1_ref[...]
        h2 = x_ref[...] + a2_ref[...]
        t1 = jnp.maximum(
            jnp.dot(h1, w1a[...], preferred_element_type=jnp.float32)
            + b1a_[...], 0.0)
        o1 = jnp.maximum(
            jnp.dot(t1, w2a[...], preferred_element_type=jnp.float32)
            + b2a_[...], 0.0)
        t2 = jnp.maximum(
            jnp.dot(h2, w1b[...], preferred_element_type=jnp.float32)
            + b1b_[...], 0.0)
        o2 = jnp.maximum(
            jnp.dot(t2, w2b[...], preferred_element_type=jnp.float32)
            + b2b_[...], 0.0)
        out = (o1 + o2) * 0.5
        mean = jnp.mean(out, axis=0, keepdims=True)
        var = jnp.mean((out - mean) ** 2, axis=0, keepdims=True)
        o_ref[...] = ((out - mean) * lax.rsqrt(var + 1e-5) * g_ref[...]
                      + be_ref[...])

    return pl.pallas_call(
        body,
        out_shape=jax.ShapeDtypeStruct((N, D), jnp.float32),
    )(x, agg1, agg2, W1a, b1a.reshape(1, D), W2a, b2a.reshape(1, D),
      W1b, b1b.reshape(1, D), W2b, b2b.reshape(1, D),
      gamma.reshape(1, D), beta.reshape(1, D))


def kernel(x, edge_index, W1a, b1a, W2a, b2a, W1b, b1b, W2b, b2b, gamma,
           beta):
    src = edge_index[0].astype(jnp.int32)
    dst = edge_index[1].astype(jnp.int32)
    zeros = jnp.zeros((NPAD, D), jnp.float32)
    agg1, agg2 = _sc_agg(x, src, dst, zeros)
    agg1 = agg1[:N]
    agg2 = agg2[:N]
    return _tc_mlp_bn(x, agg1, agg2, W1a, b1a, W2a, b2a, W1b, b1b, W2b,
                      b2b, gamma, beta)


# speedup vs baseline: 10.5868x; 1.7005x over previous
import functools

import jax
import jax.numpy as jnp
from jax import lax
from jax.experimental import pallas as pl
from jax.experimental.pallas import tpu as pltpu
from jax.experimental.pallas import tpu_sc as plsc

N = 10000
E = 320000
D = 128

NC = 2
NS = 16
BLK = 128
EPT = 20480
E_PAD = EPT * NS
BLKS_PER_TILE = EPT // BLK
SB = 32
NSB = BLKS_PER_TILE // SB
NPAD = 10240
ROWS_PER_SUB = NPAD // NS


def _sc_agg(x, g1, s1, g2, s2, zeros):
    mesh = plsc.VectorSubcoreMesh(core_axis_name="c", subcore_axis_name="s")

    @functools.partial(
        pl.kernel,
        out_type=[
            jax.ShapeDtypeStruct((NPAD, D), jnp.float32),
            jax.ShapeDtypeStruct((NPAD, D), jnp.float32),
        ],
        mesh=mesh,
        scratch_types=[
            pltpu.VMEM((SB, BLK), jnp.int32),
            pltpu.VMEM((SB, BLK), jnp.int32),
            pltpu.VMEM((BLK, D), jnp.float32),
            pltpu.VMEM((BLK, D), jnp.float32),
            pltpu.VMEM_SHARED((NPAD, D), jnp.float32),
            pltpu.SemaphoreType.DMA,
            pltpu.SemaphoreType.DMA,
        ],
    )
    def agg_kernel(x_hbm, g1_hbm, s1_hbm, g2_hbm, s2_hbm, zeros_hbm,
                   agg1_hbm, agg2_hbm,
                   gsb, ssb, rows0, rows1, acc_sp, sem0, sem1):
        c = lax.axis_index("c")
        s = lax.axis_index("s")

        pltpu.sync_copy(zeros_hbm.at[pl.ds(s * ROWS_PER_SUB, ROWS_PER_SUB)],
                        acc_sp.at[pl.ds(s * ROWS_PER_SUB, ROWS_PER_SUB)])
        plsc.subcore_barrier()

        def run_direction(gather_hbm, scatter_hbm, out_hbm):
            base_blk = s * BLKS_PER_TILE

            def sb_body(sb, _):
                row0 = base_blk + sb * SB
                pltpu.sync_copy(gather_hbm.at[pl.ds(row0, SB)], gsb)
                pltpu.sync_copy(scatter_hbm.at[pl.ds(row0, SB)], ssb)
                pltpu.async_copy(x_hbm.at[gsb.at[0]], rows0, sem0)

                def pair_body(p, carry):
                    i0 = 2 * p
                    i1 = i0 + 1
                    pltpu.async_copy(x_hbm.at[gsb.at[i1]], rows1, sem1)
                    pltpu.make_async_copy(
                        x_hbm.at[gsb.at[i0]], rows0, sem0).wait()
                    pltpu.sync_copy(rows0, acc_sp.at[ssb.at[i0]], add=True)

                    @pl.when(i1 + 1 < SB)
                    def _():
                        pltpu.async_copy(
                            x_hbm.at[gsb.at[i1 + 1]], rows0, sem0)

                    pltpu.make_async_copy(
                        x_hbm.at[gsb.at[i1]], rows1, sem1).wait()
                    pltpu.sync_copy(rows1, acc_sp.at[ssb.at[i1]], add=True)
                    return carry

                lax.fori_loop(0, SB // 2, pair_body, 0)
                return _

            lax.fori_loop(0, NSB, sb_body, 0)
            plsc.subcore_barrier()
            pltpu.sync_copy(acc_sp.at[pl.ds(s * ROWS_PER_SUB, ROWS_PER_SUB)],
                            out_hbm.at[pl.ds(s * ROWS_PER_SUB, ROWS_PER_SUB)])

        @pl.when(c == 0)
        def _():
            run_direction(g1_hbm, s1_hbm, agg1_hbm)

        @pl.when(c == 1)
        def _():
            run_direction(g2_hbm, s2_hbm, agg2_hbm)

    return agg_kernel(x, g1, s1, g2, s2, zeros)


def _tc_mlp_bn(x, agg1, agg2, W1a, b1a, W2a, b2a, W1b, b1b, W2b, b2b,
               gamma, beta):
    def body(x_ref, a1_ref, a2_ref, w1a, b1a_, w2a, b2a_, w1b, b1b_, w2b,
             b2b_, g_ref, be_ref, o_ref):
        h1 = x_ref[...] + a1_ref[...]
        h2 = x_ref[...] + a2_ref[...]
        t1 = jnp.maximum(
            jnp.dot(h1, w1a[...], preferred_element_type=jnp.float32)
            + b1a_[...], 0.0)
        o1 = jnp.maximum(
            jnp.dot(t1, w2a[...], preferred_element_type=jnp.float32)
            + b2a_[...], 0.0)
        t2 = jnp.maximum(
            jnp.dot(h2, w1b[...], preferred_element_type=jnp.float32)
            + b1b_[...], 0.0)
        o2 = jnp.maximum(
            jnp.dot(t2, w2b[...], preferred_element_type=jnp.float32)
            + b2b_[...], 0.0)
        out = (o1 + o2) * 0.5
        mean = jnp.mean(out, axis=0, keepdims=True)
        var = jnp.mean((out - mean) ** 2, axis=0, keepdims=True)
        o_ref[...] = ((out - mean) * lax.rsqrt(var + 1e-5) * g_ref[...]
                      + be_ref[...])

    full = lambda shape: pl.BlockSpec(shape, lambda i: (0,) * len(shape))
    return pl.pallas_call(
        body,
        grid=(1,),
        out_shape=jax.ShapeDtypeStruct((N, D), jnp.float32),
        in_specs=[full((N, D)), full((N, D)), full((N, D)),
                  full((D, D)), full((1, D)), full((D, D)), full((1, D)),
                  full((D, D)), full((1, D)), full((D, D)), full((1, D)),
                  full((1, D)), full((1, D))],
        out_specs=full((N, D)),
    )(x, agg1, agg2, W1a, b1a.reshape(1, D), W2a, b2a.reshape(1, D),
      W1b, b1b.reshape(1, D), W2b, b2b.reshape(1, D),
      gamma.reshape(1, D), beta.reshape(1, D))


def kernel(x, edge_index, W1a, b1a, W2a, b2a, W1b, b1b, W2b, b2b, gamma,
           beta):
    src = edge_index[0].astype(jnp.int32)
    dst = edge_index[1].astype(jnp.int32)
    npad = E_PAD - E
    pad_g = (jnp.arange(npad, dtype=jnp.int32) * 131) % N
    pad_s = N + (jnp.arange(npad, dtype=jnp.int32) % (NPAD - N))
    shape2d = (E_PAD // BLK, BLK)
    g1 = jnp.concatenate([src, pad_g]).reshape(shape2d)
    s1 = jnp.concatenate([dst, pad_s]).reshape(shape2d)
    g2 = jnp.concatenate([dst, pad_g]).reshape(shape2d)
    s2 = jnp.concatenate([src, pad_s]).reshape(shape2d)
    zeros = jnp.zeros((NPAD, D), jnp.float32)
    agg1, agg2 = _sc_agg(x, g1, s1, g2, s2, zeros)
    return _tc_mlp_bn(x, agg1, agg2, W1a, b1a, W2a, b2a, W1b, b1b, W2b,
                      b2b, gamma, beta)


# speedup vs baseline: 10.7286x; 1.0134x over previous
import functools

import jax
import jax.numpy as jnp
from jax import lax
from jax.experimental import pallas as pl
from jax.experimental.pallas import tpu as pltpu
from jax.experimental.pallas import tpu_sc as plsc

N = 10000
E = 320000
D = 128

NC = 2
NS = 16
BLK = 128
EPT = 20480
E_PAD = EPT * NS
BLKS_PER_TILE = EPT // BLK
SB = 40
NSB = BLKS_PER_TILE // SB
NPAD = 10112
ROWS_PER_SUB = NPAD // NS


def _sc_agg(x, g1, s1, g2, s2, zeros):
    mesh = plsc.VectorSubcoreMesh(core_axis_name="c", subcore_axis_name="s")

    @functools.partial(
        pl.kernel,
        out_type=[
            jax.ShapeDtypeStruct((NPAD, D), jnp.float32),
            jax.ShapeDtypeStruct((NPAD, D), jnp.float32),
        ],
        mesh=mesh,
        scratch_types=[
            pltpu.VMEM((SB, BLK), jnp.int32),
            pltpu.VMEM((SB, BLK), jnp.int32),
            pltpu.VMEM((BLK, D), jnp.float32),
            pltpu.VMEM((BLK, D), jnp.float32),
            pltpu.VMEM_SHARED((NPAD, D), jnp.float32),
            pltpu.SemaphoreType.DMA,
            pltpu.SemaphoreType.DMA,
        ],
    )
    def agg_kernel(x_hbm, g1_hbm, s1_hbm, g2_hbm, s2_hbm, zeros_hbm,
                   agg1_hbm, agg2_hbm,
                   gsb, ssb, rows0, rows1, acc_sp, sem0, sem1):
        c = lax.axis_index("c")
        s = lax.axis_index("s")

        def run_direction(gather_hbm, scatter_hbm, out_hbm):
            base_blk = s * BLKS_PER_TILE

            pltpu.sync_copy(gather_hbm.at[pl.ds(base_blk, SB)], gsb)
            pltpu.sync_copy(scatter_hbm.at[pl.ds(base_blk, SB)], ssb)
            pltpu.async_copy(x_hbm.at[gsb.at[0]], rows0, sem0)
            pltpu.sync_copy(
                zeros_hbm.at[pl.ds(s * ROWS_PER_SUB, ROWS_PER_SUB)],
                acc_sp.at[pl.ds(s * ROWS_PER_SUB, ROWS_PER_SUB)])
            plsc.subcore_barrier()

            def sb_body(sb, sb_carry):
                row0 = base_blk + sb * SB

                @pl.when(sb > 0)
                def _():
                    pltpu.sync_copy(gather_hbm.at[pl.ds(row0, SB)], gsb)
                    pltpu.sync_copy(scatter_hbm.at[pl.ds(row0, SB)], ssb)
                    pltpu.async_copy(x_hbm.at[gsb.at[0]], rows0, sem0)

                def pair_body(p, carry):
                    i0 = 2 * p
                    i1 = i0 + 1
                    pltpu.async_copy(x_hbm.at[gsb.at[i1]], rows1, sem1)
                    pltpu.make_async_copy(
                        x_hbm.at[gsb.at[i0]], rows0, sem0).wait()
                    pltpu.sync_copy(rows0, acc_sp.at[ssb.at[i0]], add=True)

                    @pl.when(i1 + 1 < SB)
                    def _():
                        pltpu.async_copy(
                            x_hbm.at[gsb.at[i1 + 1]], rows0, sem0)

                    pltpu.make_async_copy(
                        x_hbm.at[gsb.at[i1]], rows1, sem1).wait()
                    pltpu.sync_copy(rows1, acc_sp.at[ssb.at[i1]], add=True)
                    return carry

                lax.fori_loop(0, SB // 2, pair_body, 0)
                return sb_carry

            lax.fori_loop(0, NSB, sb_body, 0)
            plsc.subcore_barrier()
            pltpu.sync_copy(acc_sp.at[pl.ds(s * ROWS_PER_SUB, ROWS_PER_SUB)],
                            out_hbm.at[pl.ds(s * ROWS_PER_SUB, ROWS_PER_SUB)])

        @pl.when(c == 0)
        def _():
            run_direction(g1_hbm, s1_hbm, agg1_hbm)

        @pl.when(c == 1)
        def _():
            run_direction(g2_hbm, s2_hbm, agg2_hbm)

    return agg_kernel(x, g1, s1, g2, s2, zeros)


def _tc_mlp_bn(x, agg1, agg2, W1a, b1a, W2a, b2a, W1b, b1b, W2b, b2b,
               gamma, beta):
    def body(x_ref, a1_ref, a2_ref, w1a, b1a_, w2a, b2a_, w1b, b1b_, w2b,
             b2b_, g_ref, be_ref, o_ref):
        h1 = x_ref[...] + a1_ref[...]
        h2 = x_ref[...] + a2_ref[...]
        t1 = jnp.maximum(
            jnp.dot(h1, w1a[...], preferred_element_type=jnp.float32)
            + b1a_[...], 0.0)
        o1 = jnp.maximum(
            jnp.dot(t1, w2a[...], preferred_element_type=jnp.float32)
            + b2a_[...], 0.0)
        t2 = jnp.maximum(
            jnp.dot(h2, w1b[...], preferred_element_type=jnp.float32)
            + b1b_[...], 0.0)
        o2 = jnp.maximum(
            jnp.dot(t2, w2b[...], preferred_element_type=jnp.float32)
            + b2b_[...], 0.0)
        out = (o1 + o2) * 0.5
        mean = jnp.mean(out, axis=0, keepdims=True)
        var = jnp.mean((out - mean) ** 2, axis=0, keepdims=True)
        o_ref[...] = ((out - mean) * lax.rsqrt(var + 1e-5) * g_ref[...]
                      + be_ref[...])

    full = lambda shape: pl.BlockSpec(shape, lambda i: (0,) * len(shape))
    return pl.pallas_call(
        body,
        grid=(1,),
        out_shape=jax.ShapeDtypeStruct((N, D), jnp.float32),
        in_specs=[full((N, D)), full((N, D)), full((N, D)),
                  full((D, D)), full((1, D)), full((D, D)), full((1, D)),
                  full((D, D)), full((1, D)), full((D, D)), full((1, D)),
                  full((1, D)), full((1, D))],
        out_specs=full((N, D)),
    )(x, agg1, agg2, W1a, b1a.reshape(1, D), W2a, b2a.reshape(1, D),
      W1b, b1b.reshape(1, D), W2b, b2b.reshape(1, D),
      gamma.reshape(1, D), beta.reshape(1, D))


def kernel(x, edge_index, W1a, b1a, W2a, b2a, W1b, b1b, W2b, b2b, gamma,
           beta):
    src = edge_index[0].astype(jnp.int32)
    dst = edge_index[1].astype(jnp.int32)
    npad = E_PAD - E
    pad_g = (jnp.arange(npad, dtype=jnp.int32) * 131) % N
    pad_s = N + (jnp.arange(npad, dtype=jnp.int32) % (NPAD - N))
    shape2d = (E_PAD // BLK, BLK)
    g1 = jnp.concatenate([src, pad_g]).reshape(shape2d)
    s1 = jnp.concatenate([dst, pad_s]).reshape(shape2d)
    g2 = jnp.concatenate([dst, pad_g]).reshape(shape2d)
    s2 = jnp.concatenate([src, pad_s]).reshape(shape2d)
    zeros = jnp.zeros((NPAD, D), jnp.float32)
    agg1, agg2 = _sc_agg(x, g1, s1, g2, s2, zeros)
    return _tc_mlp_bn(x, agg1, agg2, W1a, b1a, W2a, b2a, W1b, b1b, W2b,
                      b2b, gamma, beta)


# speedup vs baseline: 11.0016x; 1.0254x over previous
import functools

import jax
import jax.numpy as jnp
from jax import lax
from jax.experimental import pallas as pl
from jax.experimental.pallas import tpu as pltpu
from jax.experimental.pallas import tpu_sc as plsc

N = 10000
E = 320000
D = 128

NC = 2
NS = 16
EPT = E // NS
BLK = 128
NBLK = EPT // BLK
TAIL = EPT - NBLK * BLK
SB = 52
NSB = NBLK // SB
SBE = SB * BLK
ROWS_A = 632
ROWS_B = N - 15 * ROWS_A


def _sc_agg(x, src, dst, zeros):
    mesh = plsc.VectorSubcoreMesh(core_axis_name="c", subcore_axis_name="s")

    @functools.partial(
        pl.kernel,
        out_type=[
            jax.ShapeDtypeStruct((N, D), jnp.float32),
            jax.ShapeDtypeStruct((N, D), jnp.float32),
        ],
        mesh=mesh,
        scratch_types=[
            pltpu.VMEM((SBE,), jnp.int32),
            pltpu.VMEM((SBE,), jnp.int32),
            pltpu.VMEM((BLK, D), jnp.float32),
            pltpu.VMEM((BLK, D), jnp.float32),
            pltpu.VMEM((TAIL,), jnp.int32),
            pltpu.VMEM((TAIL,), jnp.int32),
            pltpu.VMEM((TAIL, D), jnp.float32),
            pltpu.VMEM_SHARED((N, D), jnp.float32),
            pltpu.SemaphoreType.DMA,
            pltpu.SemaphoreType.DMA,
        ],
    )
    def agg_kernel(x_hbm, src_hbm, dst_hbm, zeros_hbm, agg1_hbm, agg2_hbm,
                   gsb, ssb, rows0, rows1, gt, st, rows_t, acc_sp,
                   sem0, sem1):
        c = lax.axis_index("c")
        s = lax.axis_index("s")

        def rows_copy(src_ref, dst_ref):
            @pl.when(s < 15)
            def _():
                pltpu.sync_copy(src_ref.at[pl.ds(s * ROWS_A, ROWS_A)],
                                dst_ref.at[pl.ds(s * ROWS_A, ROWS_A)])

            @pl.when(s == 15)
            def _():
                pltpu.sync_copy(src_ref.at[pl.ds(15 * ROWS_A, ROWS_B)],
                                dst_ref.at[pl.ds(15 * ROWS_A, ROWS_B)])

        def run_direction(gather_hbm, scatter_hbm, out_hbm):
            base_e = s * EPT

            pltpu.sync_copy(gather_hbm.at[pl.ds(base_e, SBE)], gsb)
            pltpu.sync_copy(scatter_hbm.at[pl.ds(base_e, SBE)], ssb)
            pltpu.async_copy(x_hbm.at[gsb.at[pl.ds(0, BLK)]], rows0, sem0)
            rows_copy(zeros_hbm, acc_sp)
            plsc.subcore_barrier()

            def sb_body(sb, sb_carry):
                @pl.when(sb > 0)
                def _():
                    off = base_e + sb * SBE
                    pltpu.sync_copy(gather_hbm.at[pl.ds(off, SBE)], gsb)
                    pltpu.sync_copy(scatter_hbm.at[pl.ds(off, SBE)], ssb)
                    pltpu.async_copy(
                        x_hbm.at[gsb.at[pl.ds(0, BLK)]], rows0, sem0)

                def pair_body(p, carry):
                    e0 = 2 * p * BLK
                    e1 = e0 + BLK
                    pltpu.async_copy(
                        x_hbm.at[gsb.at[pl.ds(e1, BLK)]], rows1, sem1)
                    pltpu.make_async_copy(
                        x_hbm.at[gsb.at[pl.ds(e0, BLK)]], rows0, sem0).wait()
                    pltpu.sync_copy(
                        rows0, acc_sp.at[ssb.at[pl.ds(e0, BLK)]], add=True)

                    @pl.when(e1 + BLK < SBE)
                    def _():
                        pltpu.async_copy(
                            x_hbm.at[gsb.at[pl.ds(e1 + BLK, BLK)]],
                            rows0, sem0)

                    pltpu.make_async_copy(
                        x_hbm.at[gsb.at[pl.ds(e1, BLK)]], rows1, sem1).wait()
                    pltpu.sync_copy(
                        rows1, acc_sp.at[ssb.at[pl.ds(e1, BLK)]], add=True)
                    return carry

                lax.fori_loop(0, SB // 2, pair_body, 0)
                return sb_carry

            lax.fori_loop(0, NSB, sb_body, 0)

            toff = base_e + NBLK * BLK
            pltpu.sync_copy(gather_hbm.at[pl.ds(toff, TAIL)], gt)
            pltpu.sync_copy(scatter_hbm.at[pl.ds(toff, TAIL)], st)
            pltpu.async_copy(x_hbm.at[gt], rows_t, sem0).wait()
            pltpu.sync_copy(rows_t, acc_sp.at[st], add=True)

            plsc.subcore_barrier()
            rows_copy(acc_sp, out_hbm)

        @pl.when(c == 0)
        def _():
            run_direction(src_hbm, dst_hbm, agg1_hbm)

        @pl.when(c == 1)
        def _():
            run_direction(dst_hbm, src_hbm, agg2_hbm)

    return agg_kernel(x, src, dst, zeros)


def _tc_mlp_bn(x, agg1, agg2, W1a, b1a, W2a, b2a, W1b, b1b, W2b, b2b,
               gamma, beta):
    def body(x_ref, a1_ref, a2_ref, w1a, b1a_, w2a, b2a_, w1b, b1b_, w2b,
             b2b_, g_ref, be_ref, o_ref):
        h1 = x_ref[...] + a1_ref[...]
        h2 = x_ref[...] + a2_ref[...]
        t1 = jnp.maximum(
            jnp.dot(h1, w1a[...], preferred_element_type=jnp.float32)
            + b1a_[...], 0.0)
        o1 = jnp.maximum(
            jnp.dot(t1, w2a[...], preferred_element_type=jnp.float32)
            + b2a_[...], 0.0)
        t2 = jnp.maximum(
            jnp.dot(h2, w1b[...], preferred_element_type=jnp.float32)
            + b1b_[...], 0.0)
        o2 = jnp.maximum(
            jnp.dot(t2, w2b[...], preferred_element_type=jnp.float32)
            + b2b_[...], 0.0)
        out = (o1 + o2) * 0.5
        mean = jnp.mean(out, axis=0, keepdims=True)
        var = jnp.mean((out - mean) ** 2, axis=0, keepdims=True)
        o_ref[...] = ((out - mean) * lax.rsqrt(var + 1e-5) * g_ref[...]
                      + be_ref[...])

    return pl.pallas_call(
        body,
        out_shape=jax.ShapeDtypeStruct((N, D), jnp.float32),
    )(x, agg1, agg2, W1a, b1a.reshape(1, D), W2a, b2a.reshape(1, D),
      W1b, b1b.reshape(1, D), W2b, b2b.reshape(1, D),
      gamma.reshape(1, D), beta.reshape(1, D))


def kernel(x, edge_index, W1a, b1a, W2a, b2a, W1b, b1b, W2b, b2b, gamma,
           beta):
    src = edge_index[0].astype(jnp.int32)
    dst = edge_index[1].astype(jnp.int32)
    zeros = jnp.zeros((N, D), jnp.float32)
    agg1, agg2 = _sc_agg(x, src, dst, zeros)
    return _tc_mlp_bn(x, agg1, agg2, W1a, b1a, W2a, b2a, W1b, b1b, W2b,
                      b2b, gamma, beta)
